# baseline (speedup 1.0000x reference)
import functools

import jax
import jax.numpy as jnp
from jax.experimental import pallas as pl
from jax.experimental.pallas import tpu as pltpu

N = 100000
D = 128
S = 256
DI = 256
B = 2000
NB = N // B
W = 64
NW = 5
ACC_R = 576


def _fused_kernel(los, his,
                  xu, iu, xv, iv, xy, iy,
                  wgu, bgu, wgv, bgv, wgy, bgy, wnet, bnet,
                  out, accu, accv, accy):
    i = pl.program_id(0)

    @pl.when(i == 0)
    def _init():
        accu[...] = jnp.zeros_like(accu)
        accv[...] = jnp.zeros_like(accv)
        accy[...] = jnp.zeros_like(accy)

    for p, (xref, iref, wg, bg, acc) in enumerate((
            (xu, iu, wgu, bgu, accu),
            (xv, iv, wgv, bgv, accv),
            (xy, iy, wgy, bgy, accy))):
        x = xref[...]
        gate = jax.nn.sigmoid(
            jnp.dot(x.astype(jnp.bfloat16), wg[...].astype(jnp.bfloat16),
                    preferred_element_type=jnp.float32)
            + bg[...])
        e = jnp.exp(gate)
        payload = jnp.concatenate([e, e * x], axis=1).astype(jnp.bfloat16)
        base = (los[i, p] // 8) * 8
        hi = his[i, p]
        d = iref[0] - base
        iot = jax.lax.broadcasted_iota(jnp.int32, (W, B), 0)
        for k in range(NW):
            def _window(k=k):
                onehot_t = (iot == d - (W * k)).astype(jnp.bfloat16)
                part = jnp.dot(onehot_t, payload,
                               preferred_element_type=jnp.float32)
                r = pl.ds(base + W * k, W)
                acc[r, :] += part
            if k == 0:
                _window()
            else:
                pl.when(base + W * k <= hi)(_window)

    @pl.when(i == NB - 1)
    def _finish():
        res = bnet[...]
        for k, acc in enumerate((accu, accv, accy)):
            seg_e = acc[:S, :D]
            seg_ex = acc[:S, D:]
            h = seg_ex / (seg_e + 1e-16)
            res = res + jnp.dot(h, wnet[k],
                                preferred_element_type=jnp.float32)
        out[...] = res


@functools.partial(jax.jit, static_argnames=())
def kernel(x_u, x_v, x_y, index_u, index_v, index_y,
           Wg_u, bg_u, Wg_v, bg_v, Wg_y, bg_y, W_net, b_net):
    iu = index_u.astype(jnp.int32).reshape(NB, 1, B)
    iv = index_v.astype(jnp.int32).reshape(NB, 1, B)
    iy = index_y.astype(jnp.int32).reshape(NB, 1, B)
    los = jnp.stack([iu[:, 0, 0], iv[:, 0, 0], iy[:, 0, 0]], axis=1)
    his = jnp.stack([iu[:, 0, B - 1], iv[:, 0, B - 1], iy[:, 0, B - 1]],
                    axis=1)
    wnet = W_net.reshape(3, D, DI)

    x_spec = pl.BlockSpec((B, D), lambda i, *_: (i, 0))
    i_spec = pl.BlockSpec((1, 1, B), lambda i, *_: (i, 0, 0))
    w_spec = pl.BlockSpec((D, D), lambda i, *_: (0, 0))
    b_spec = pl.BlockSpec((1, D), lambda i, *_: (0, 0))

    out = pl.pallas_call(
        _fused_kernel,
        grid_spec=pltpu.PrefetchScalarGridSpec(
            num_scalar_prefetch=2,
            grid=(NB,),
            in_specs=[
                x_spec, i_spec, x_spec, i_spec, x_spec, i_spec,
                w_spec, b_spec, w_spec, b_spec, w_spec, b_spec,
                pl.BlockSpec((3, D, DI), lambda i, *_: (0, 0, 0)),
                pl.BlockSpec((1, DI), lambda i, *_: (0, 0)),
            ],
            out_specs=pl.BlockSpec((S, DI), lambda i, *_: (0, 0)),
            scratch_shapes=[pltpu.VMEM((ACC_R, 2 * D), jnp.float32)] * 3,
        ),
        out_shape=jax.ShapeDtypeStruct((S, DI), jnp.float32),
    )(los, his, x_u, iu, x_v, iv, x_y, iy,
      Wg_u, bg_u.reshape(1, D), Wg_v, bg_v.reshape(1, D),
      Wg_y, bg_y.reshape(1, D), wnet, b_net.reshape(1, DI))
    return out


# speedup vs baseline: 1.3361x; 1.3361x over previous
import functools

import jax
import jax.numpy as jnp
from jax.experimental import pallas as pl
from jax.experimental.pallas import tpu as pltpu

N = 100000
D = 128
S = 256
DI = 256
B = 2000
NB = N // B


def _fused_kernel(xu, iu, xv, iv, xy, iy,
                  wgu, bgu, wgv, bgv, wgy, bgy, wnet, bnet,
                  out, accu, accv, accy):
    i = pl.program_id(0)

    @pl.when(i == 0)
    def _init():
        accu[...] = jnp.zeros_like(accu)
        accv[...] = jnp.zeros_like(accv)
        accy[...] = jnp.zeros_like(accy)

    for xref, iref, wg, bg, acc in (
            (xu, iu, wgu, bgu, accu),
            (xv, iv, wgv, bgv, accv),
            (xy, iy, wgy, bgy, accy)):
        xb = xref[...].astype(jnp.bfloat16)
        z = jnp.dot(xb, wg[...].astype(jnp.bfloat16),
                    preferred_element_type=jnp.float32) + bg[...]
        gate = 0.5 + 0.5 * jnp.tanh(0.5 * z)
        eb = jnp.exp(gate).astype(jnp.bfloat16)
        payload = jnp.concatenate([eb, eb * xb], axis=1)
        idx = iref[0]
        onehot_t = (jax.lax.broadcasted_iota(jnp.int32, (S, B), 0)
                    == idx).astype(jnp.bfloat16)
        acc[...] += jnp.dot(onehot_t, payload,
                            preferred_element_type=jnp.float32)

    @pl.when(i == NB - 1)
    def _finish():
        res = bnet[...]
        for k, acc in enumerate((accu, accv, accy)):
            seg_e = acc[:, :D]
            seg_ex = acc[:, D:]
            h = seg_ex / (seg_e + 1e-16)
            res = res + jnp.dot(h, wnet[k],
                                preferred_element_type=jnp.float32)
        out[...] = res


@functools.partial(jax.jit, static_argnames=())
def kernel(x_u, x_v, x_y, index_u, index_v, index_y,
           Wg_u, bg_u, Wg_v, bg_v, Wg_y, bg_y, W_net, b_net):
    iu = index_u.astype(jnp.int32).reshape(NB, 1, B)
    iv = index_v.astype(jnp.int32).reshape(NB, 1, B)
    iy = index_y.astype(jnp.int32).reshape(NB, 1, B)
    wnet = W_net.reshape(3, D, DI)

    x_spec = pl.BlockSpec((B, D), lambda i: (i, 0))
    i_spec = pl.BlockSpec((1, 1, B), lambda i: (i, 0, 0))
    w_spec = pl.BlockSpec((D, D), lambda i: (0, 0))
    b_spec = pl.BlockSpec((1, D), lambda i: (0, 0))

    out = pl.pallas_call(
        _fused_kernel,
        grid=(NB,),
        in_specs=[
            x_spec, i_spec, x_spec, i_spec, x_spec, i_spec,
            w_spec, b_spec, w_spec, b_spec, w_spec, b_spec,
            pl.BlockSpec((3, D, DI), lambda i: (0, 0, 0)),
            pl.BlockSpec((1, DI), lambda i: (0, 0)),
        ],
        out_specs=pl.BlockSpec((S, DI), lambda i: (0, 0)),
        out_shape=jax.ShapeDtypeStruct((S, DI), jnp.float32),
        scratch_shapes=[pltpu.VMEM((S, 2 * D), jnp.float32)] * 3,
    )(x_u, iu, x_v, iv, x_y, iy,
      Wg_u, bg_u.reshape(1, D), Wg_v, bg_v.reshape(1, D),
      Wg_y, bg_y.reshape(1, D), wnet, b_net.reshape(1, DI))
    return out


# speedup vs baseline: 1.3384x; 1.0017x over previous
import functools

import jax
import jax.numpy as jnp
from jax.experimental import pallas as pl
from jax.experimental.pallas import tpu as pltpu

N = 100000
D = 128
S = 256
DI = 256
B = 2000
NB = N // B


def _fused_kernel(xu, iu, xv, iv, xy, iy,
                  wgu, bgu, wgv, bgv, wgy, bgy, wnet, bnet,
                  out, accu, accv, accy):
    i = pl.program_id(0)

    @pl.when(i == 0)
    def _init():
        accu[...] = jnp.zeros_like(accu)
        accv[...] = jnp.zeros_like(accv)
        accy[...] = jnp.zeros_like(accy)

    for xref, iref, wg, bg, acc in (
            (xu, iu, wgu, bgu, accu),
            (xv, iv, wgv, bgv, accv),
            (xy, iy, wgy, bgy, accy)):
        xb = xref[...].astype(jnp.bfloat16)
        z = jnp.dot(xb, wg[...].astype(jnp.bfloat16),
                    preferred_element_type=jnp.float32) + bg[...]
        c1 = 0.5 * 1.4426950408889634
        eb = jnp.exp2(c1 * jnp.tanh(0.5 * z)).astype(jnp.bfloat16)
        payload = jnp.concatenate([eb, eb * xb], axis=1)
        idx = iref[0]
        onehot_t = (jax.lax.broadcasted_iota(jnp.int32, (S, B), 0)
                    == idx).astype(jnp.bfloat16)
        acc[...] += jnp.dot(onehot_t, payload,
                            preferred_element_type=jnp.float32)

    @pl.when(i == NB - 1)
    def _finish():
        res = bnet[...]
        for k, acc in enumerate((accu, accv, accy)):
            seg_e = acc[:, :D]
            seg_ex = acc[:, D:]
            h = seg_ex / (seg_e + 1e-16)
            res = res + jnp.dot(h, wnet[k],
                                preferred_element_type=jnp.float32)
        out[...] = res


@functools.partial(jax.jit, static_argnames=())
def kernel(x_u, x_v, x_y, index_u, index_v, index_y,
           Wg_u, bg_u, Wg_v, bg_v, Wg_y, bg_y, W_net, b_net):
    iu = index_u.astype(jnp.int32).reshape(NB, 1, B)
    iv = index_v.astype(jnp.int32).reshape(NB, 1, B)
    iy = index_y.astype(jnp.int32).reshape(NB, 1, B)
    wnet = W_net.reshape(3, D, DI)

    x_spec = pl.BlockSpec((B, D), lambda i: (i, 0))
    i_spec = pl.BlockSpec((1, 1, B), lambda i: (i, 0, 0))
    w_spec = pl.BlockSpec((D, D), lambda i: (0, 0))
    b_spec = pl.BlockSpec((1, D), lambda i: (0, 0))

    out = pl.pallas_call(
        _fused_kernel,
        grid=(NB,),
        in_specs=[
            x_spec, i_spec, x_spec, i_spec, x_spec, i_spec,
            w_spec, b_spec, w_spec, b_spec, w_spec, b_spec,
            pl.BlockSpec((3, D, DI), lambda i: (0, 0, 0)),
            pl.BlockSpec((1, DI), lambda i: (0, 0)),
        ],
        out_specs=pl.BlockSpec((S, DI), lambda i: (0, 0)),
        out_shape=jax.ShapeDtypeStruct((S, DI), jnp.float32),
        scratch_shapes=[pltpu.VMEM((S, 2 * D), jnp.float32)] * 3,
    )(x_u, iu, x_v, iv, x_y, iy,
      Wg_u, bg_u.reshape(1, D), Wg_v, bg_v.reshape(1, D),
      Wg_y, bg_y.reshape(1, D), wnet, b_net.reshape(1, DI))
    return out
